# baseline (speedup 1.0000x reference)
import functools

import jax
import jax.numpy as jnp
from jax import lax
from jax.experimental import pallas as pl
from jax.experimental.pallas import tpu as pltpu
from jax.experimental.pallas import tpu_sc as plsc

N, M, U, D = 32, 4096, 2048, 256
CH = 256


@functools.lru_cache(maxsize=1)
def _build_sc_kernel():
    info = plsc.get_sparse_core_info()
    nc = info.num_cores
    mesh = plsc.VectorSubcoreMesh(core_axis_name="c", subcore_axis_name="s")

    @functools.partial(
        pl.kernel,
        out_type=jax.ShapeDtypeStruct((N, M, D), jnp.float32),
        mesh=mesh,
        scratch_types=[pltpu.VMEM((16,), jnp.int32)],
        compiler_params=pltpu.CompilerParams(use_tc_tiling_on_sc=False),
    )
    def k(var_hbm, upd_hbm, idx_hbm, out_hbm, idx_v):
        wid = lax.axis_index("s") * nc + lax.axis_index("c")
        pltpu.sync_copy(idx_hbm.at[wid], idx_v)
        v = idx_v[...]
        off = v[0]
        ln = v[1]

        pltpu.sync_copy(var_hbm.at[wid], out_hbm.at[wid])

        nfull = ln // CH

        def body(kk, c):
            pltpu.sync_copy(
                upd_hbm.at[wid, pl.ds(kk * CH, CH)],
                out_hbm.at[wid, pl.ds(off + kk * CH, CH)],
            )
            return c

        lax.fori_loop(0, nfull, body, 0)

        rem = ln - nfull * CH

        @pl.when(jnp.logical_and(rem > 0, ln >= CH))
        def _():
            pltpu.sync_copy(
                upd_hbm.at[wid, pl.ds(ln - CH, CH)],
                out_hbm.at[wid, pl.ds(off + ln - CH, CH)],
            )

        @pl.when(jnp.logical_and(rem > 0, ln < CH))
        def _():
            def body1(r, c):
                pltpu.sync_copy(
                    upd_hbm.at[wid, pl.ds(r, 1)],
                    out_hbm.at[wid, pl.ds(off + r, 1)],
                )
                return c

            lax.fori_loop(0, ln, body1, 0)

    return k


def kernel(varRef, indice, updates, mask, reduce, axis):
    idx = indice.astype(jnp.int32)
    off = jnp.clip(idx[:, 0], 0, M)
    ln = jnp.clip(idx[:, 1], 0, M - off)
    ln = jnp.where(mask, ln, 0)
    idx16 = jnp.zeros((N, 16), jnp.int32)
    idx16 = idx16.at[:, 0].set(off).at[:, 1].set(ln)
    return _build_sc_kernel()(varRef, updates, idx16)


# speedup vs baseline: 12.5311x; 12.5311x over previous
import functools

import jax
import jax.numpy as jnp
from jax import lax
from jax.experimental import pallas as pl
from jax.experimental.pallas import tpu as pltpu
from jax.experimental.pallas import tpu_sc as plsc

N, M, U, D = 32, 4096, 2048, 256
CH = 128
NCH = M // CH


@functools.lru_cache(maxsize=1)
def _build_sc_kernel():
    info = plsc.get_sparse_core_info()
    nc = info.num_cores
    mesh = plsc.VectorSubcoreMesh(core_axis_name="c", subcore_axis_name="s")

    @functools.partial(
        pl.kernel,
        out_type=jax.ShapeDtypeStruct((N, M, D), jnp.float32),
        mesh=mesh,
        scratch_types=[
            pltpu.VMEM((16,), jnp.int32),
            pltpu.VMEM((2, CH, D), jnp.float32),
            pltpu.SemaphoreType.DMA,
            pltpu.SemaphoreType.DMA,
            pltpu.SemaphoreType.DMA,
            pltpu.SemaphoreType.DMA,
        ],
        compiler_params=pltpu.CompilerParams(use_tc_tiling_on_sc=False),
    )
    def k(var_hbm, upd_hbm, idx_hbm, out_hbm, idx_v, buf, si0, si1, so0, so1):
        wid = lax.axis_index("s") * nc + lax.axis_index("c")
        pltpu.sync_copy(idx_hbm.at[wid], idx_v)
        v = idx_v[...]
        off = v[0]
        ln = v[1]

        sin = (si0, si1)
        sout = (so0, so1)

        def gather(kk):
            return pltpu.make_async_copy(
                var_hbm.at[wid, pl.ds(kk * CH, CH)], buf.at[kk % 2], sin[kk % 2]
            )

        def scatter(kk):
            return pltpu.make_async_copy(
                buf.at[kk % 2], out_hbm.at[wid, pl.ds(kk * CH, CH)], sout[kk % 2]
            )

        gather(0).start()
        for kk in range(NCH):
            gather(kk).wait()
            scatter(kk).start()
            if kk + 1 < NCH:
                if kk >= 1:
                    scatter(kk - 1).wait()
                gather(kk + 1).start()
        scatter(NCH - 2).wait()
        scatter(NCH - 1).wait()

        nfull = ln // CH

        def body(kk, c):
            pltpu.sync_copy(upd_hbm.at[wid, pl.ds(kk * CH, CH)], buf.at[0])
            pltpu.sync_copy(buf.at[0], out_hbm.at[wid, pl.ds(off + kk * CH, CH)])
            return c

        lax.fori_loop(0, nfull, body, 0)

        base = nfull * CH
        rem = ln - base
        sz = CH // 2
        while sz >= 1:
            s = sz

            @pl.when(rem >= s)
            def _():
                pltpu.sync_copy(
                    upd_hbm.at[wid, pl.ds(base, s)], buf.at[0, pl.ds(0, s)]
                )
                pltpu.sync_copy(
                    buf.at[0, pl.ds(0, s)], out_hbm.at[wid, pl.ds(off + base, s)]
                )

            base = base + jnp.where(rem >= s, s, 0)
            rem = rem - jnp.where(rem >= s, s, 0)
            sz //= 2

    return k


def kernel(varRef, indice, updates, mask, reduce, axis):
    idx = indice.astype(jnp.int32)
    off = jnp.clip(idx[:, 0], 0, M)
    ln = jnp.clip(idx[:, 1], 0, M - off)
    ln = jnp.where(mask, ln, 0)
    idx16 = jnp.zeros((N, 16), jnp.int32)
    idx16 = idx16.at[:, 0].set(off).at[:, 1].set(ln)
    return _build_sc_kernel()(varRef, updates, idx16)


# speedup vs baseline: 13.7071x; 1.0938x over previous
import functools

import jax
import jax.numpy as jnp
from jax import lax
from jax.experimental import pallas as pl
from jax.experimental.pallas import tpu as pltpu
from jax.experimental.pallas import tpu_sc as plsc

N, M, U, D = 32, 4096, 2048, 256
CH = 128
NCH = M // CH
R = 3


@functools.lru_cache(maxsize=1)
def _build_sc_kernel():
    info = plsc.get_sparse_core_info()
    nc = info.num_cores
    mesh = plsc.VectorSubcoreMesh(core_axis_name="c", subcore_axis_name="s")

    @functools.partial(
        pl.kernel,
        out_type=jax.ShapeDtypeStruct((N, M, D), jnp.float32),
        mesh=mesh,
        scratch_types=[
            pltpu.VMEM((16,), jnp.int32),
            pltpu.VMEM((R, CH, D), jnp.float32),
            [pltpu.SemaphoreType.DMA] * R,
            [pltpu.SemaphoreType.DMA] * R,
        ],
        compiler_params=pltpu.CompilerParams(use_tc_tiling_on_sc=False),
    )
    def k(var_hbm, upd_hbm, idx_hbm, out_hbm, idx_v, buf, sg, ss):
        wid = lax.axis_index("s") * nc + lax.axis_index("c")
        pltpu.sync_copy(idx_hbm.at[wid], idx_v)
        v = idx_v[...]
        off = v[0]
        ln = v[1]
        end = off + ln

        def gather(kk):
            p = kk % R
            base = kk * CH
            inside = jnp.logical_and(off <= base, base + CH <= end)

            @pl.when(inside)
            def _():
                pltpu.make_async_copy(
                    upd_hbm.at[wid, pl.ds(jnp.maximum(base - off, 0), CH)],
                    buf.at[p],
                    sg[p],
                ).start()

            @pl.when(jnp.logical_not(inside))
            def _():
                pltpu.make_async_copy(
                    var_hbm.at[wid, pl.ds(base, CH)], buf.at[p], sg[p]
                ).start()

        def gather_wait(kk):
            p = kk % R
            pltpu.make_async_copy(
                var_hbm.at[wid, pl.ds(kk * CH, CH)], buf.at[p], sg[p]
            ).wait()

        def scatter(kk):
            p = kk % R
            return pltpu.make_async_copy(
                buf.at[p], out_hbm.at[wid, pl.ds(kk * CH, CH)], ss[p]
            )

        gather(0)
        for kk in range(NCH):
            if kk + 1 < NCH:
                if kk + 1 >= R:
                    scatter(kk + 1 - R).wait()
                gather(kk + 1)
            gather_wait(kk)
            scatter(kk).start()
        for kk in range(NCH - R, NCH):
            scatter(kk).wait()

        def copy_seg(seg_lo, seg_hi):
            base = seg_lo
            rem = jnp.maximum(seg_hi - seg_lo, 0)
            sz = CH // 2
            while sz >= 1:
                s = sz

                @pl.when(rem >= s)
                def _():
                    pltpu.sync_copy(
                        upd_hbm.at[wid, pl.ds(jnp.maximum(base - off, 0), s)],
                        buf.at[0, pl.ds(0, s)],
                    )
                    pltpu.sync_copy(
                        buf.at[0, pl.ds(0, s)],
                        out_hbm.at[wid, pl.ds(base, s)],
                    )

                step = jnp.where(rem >= s, s, 0)
                base = base + step
                rem = rem - step
                sz //= 2

        k0 = off // CH
        k0_end = k0 * CH + CH
        inside0 = jnp.logical_and(off <= k0 * CH, k0_end <= end)

        @pl.when(jnp.logical_not(inside0))
        def _():
            copy_seg(off, jnp.minimum(k0_end, end))

        k1 = end // CH
        k1_start = k1 * CH
        inside1 = jnp.logical_and(off <= k1_start, k1_start + CH <= end)

        @pl.when(jnp.logical_and(jnp.logical_not(inside1), k1 != k0))
        def _():
            copy_seg(jnp.maximum(k1_start, off), end)

    return k


def kernel(varRef, indice, updates, mask, reduce, axis):
    idx = indice.astype(jnp.int32)
    off = jnp.clip(idx[:, 0], 0, M)
    ln = jnp.clip(idx[:, 1], 0, M - off)
    ln = jnp.where(mask, ln, 0)
    idx16 = jnp.zeros((N, 16), jnp.int32)
    idx16 = idx16.at[:, 0].set(off).at[:, 1].set(ln)
    return _build_sc_kernel()(varRef, updates, idx16)


# speedup vs baseline: 41.8998x; 3.0568x over previous
import functools

import jax
import jax.numpy as jnp
from jax import lax
from jax.experimental import pallas as pl
from jax.experimental.pallas import tpu as pltpu
from jax.experimental.pallas import tpu_sc as plsc

N, M, U, D = 32, 4096, 2048, 256
CH = 64
NCH = M // CH
UCH = U // CH
R = 6
G = 3


@functools.lru_cache(maxsize=1)
def _build_sc_kernel():
    info = plsc.get_sparse_core_info()
    nc = info.num_cores
    mesh = plsc.VectorSubcoreMesh(core_axis_name="c", subcore_axis_name="s")

    @functools.partial(
        pl.kernel,
        out_type=jax.ShapeDtypeStruct((N * M, D), jnp.float32),
        mesh=mesh,
        scratch_types=[
            pltpu.VMEM((8, 128), jnp.int32),
            pltpu.VMEM((R, CH, D), jnp.float32),
            [pltpu.VMEM((CH,), jnp.int32)] * R,
            [pltpu.VMEM((CH,), jnp.int32)] * R,
            [pltpu.SemaphoreType.DMA] * R,
            [pltpu.SemaphoreType.DMA] * R,
        ],
    )
    def k(var_hbm, upd_hbm, idx_hbm, out_hbm, idx_v, buf, sidx, didx, sg, ss):
        wid = lax.axis_index("s") * nc + lax.axis_index("c")
        pltpu.sync_copy(idx_hbm.at[wid], idx_v)
        v = idx_v[0, pl.ds(0, 16)]
        off = v[0]
        ln = v[1]
        end = off + ln
        vbase = wid * M
        ubase = wid * U

        def al(x):
            return pl.multiple_of(x, 8)

        def copy_cond(kk):
            b = kk * CH
            return jnp.logical_not(jnp.logical_and(off <= b, b + CH <= end))

        def g_var(kk):
            p = kk % R
            return pltpu.make_async_copy(
                var_hbm.at[pl.ds(al(vbase + kk * CH), CH)], buf.at[p], sg[p]
            )

        def s_out(kk):
            p = kk % R
            return pltpu.make_async_copy(
                buf.at[p], out_hbm.at[pl.ds(al(vbase + kk * CH), CH)], ss[p]
            )

        for kk in range(NCH + G):
            if kk < NCH:
                if kk >= R:

                    @pl.when(copy_cond(kk - R))
                    def _():
                        s_out(kk - R).wait()

                @pl.when(copy_cond(kk))
                def _():
                    g_var(kk).start()

            if kk >= G:
                j = kk - G

                @pl.when(copy_cond(j))
                def _():
                    g_var(j).wait()
                    s_out(j).start()

        for j in range(NCH - R, NCH):

            @pl.when(copy_cond(j))
            def _():
                s_out(j).wait()

        nch = (ln + CH - 1) // CH
        iota = lax.iota(jnp.int32, 16)

        def g_upd(t):
            p = t % R
            return pltpu.make_async_copy(upd_hbm.at[sidx[p]], buf.at[p], sg[p])

        def s_upd(t):
            p = t % R
            return pltpu.make_async_copy(buf.at[p], out_hbm.at[didx[p]], ss[p])

        for t in range(UCH + 2):
            if t < UCH:
                if t >= R:

                    @pl.when(t - R < nch)
                    def _():
                        s_upd(t - R).wait()

                @pl.when(t < nch)
                def _():
                    p = t % R
                    for b in range(CH // 16):
                        q = jnp.minimum(t * CH + b * 16 + iota, ln - 1)
                        sidx[p][pl.ds(b * 16, 16)] = ubase + q
                        didx[p][pl.ds(b * 16, 16)] = vbase + off + q
                    g_upd(t).start()

            if t >= 2:
                j = t - 2

                @pl.when(j < nch)
                def _():
                    g_upd(j).wait()
                    s_upd(j).start()

        for j in range(UCH - R, UCH):

            @pl.when(j < nch)
            def _():
                s_upd(j).wait()

    return k


def kernel(varRef, indice, updates, mask, reduce, axis):
    idx = indice.astype(jnp.int32)
    off = jnp.clip(idx[:, 0], 0, M)
    ln = jnp.clip(idx[:, 1], 0, M - off)
    ln = jnp.where(mask, ln, 0)
    idx3 = jnp.zeros((N, 8, 128), jnp.int32)
    idx3 = idx3.at[:, 0, 0].set(off).at[:, 0, 1].set(ln)
    out = _build_sc_kernel()(
        varRef.reshape(N * M, D), updates.reshape(N * U, D), idx3
    )
    return out.reshape(N, M, D)


# speedup vs baseline: 42.6489x; 1.0179x over previous
import functools

import jax
import jax.numpy as jnp
from jax import lax
from jax.experimental import pallas as pl
from jax.experimental.pallas import tpu as pltpu
from jax.experimental.pallas import tpu_sc as plsc

N, M, U, D = 32, 4096, 2048, 256
CH = 64
NCH = M // CH
UCH = U // CH
R = 7
G = 4


@functools.lru_cache(maxsize=1)
def _build_sc_kernel():
    info = plsc.get_sparse_core_info()
    nc = info.num_cores
    mesh = plsc.VectorSubcoreMesh(core_axis_name="c", subcore_axis_name="s")

    @functools.partial(
        pl.kernel,
        out_type=jax.ShapeDtypeStruct((N * M, D), jnp.float32),
        mesh=mesh,
        scratch_types=[
            pltpu.VMEM((8, 128), jnp.int32),
            pltpu.VMEM((R, CH, D), jnp.float32),
            [pltpu.VMEM((CH,), jnp.int32)] * R,
            [pltpu.VMEM((CH,), jnp.int32)] * R,
            [pltpu.SemaphoreType.DMA] * R,
            [pltpu.SemaphoreType.DMA] * R,
        ],
    )
    def k(var_hbm, upd_hbm, idx_hbm, out_hbm, idx_v, buf, sidx, didx, sg, ss):
        wid = lax.axis_index("s") * nc + lax.axis_index("c")
        pltpu.sync_copy(idx_hbm.at[wid], idx_v)
        v = idx_v[0, pl.ds(0, 16)]
        off = v[0]
        ln = v[1]
        end = off + ln
        vbase = wid * M
        ubase = wid * U

        def al(x):
            return pl.multiple_of(x, 8)

        def copy_cond(kk):
            b = kk * CH
            return jnp.logical_not(jnp.logical_and(off <= b, b + CH <= end))

        def g_var(kk):
            p = kk % R
            return pltpu.make_async_copy(
                var_hbm.at[pl.ds(al(vbase + kk * CH), CH)], buf.at[p], sg[p]
            )

        def s_out(kk):
            p = kk % R
            return pltpu.make_async_copy(
                buf.at[p], out_hbm.at[pl.ds(al(vbase + kk * CH), CH)], ss[p]
            )

        for kk in range(NCH + G):
            if kk < NCH:
                if kk >= R:

                    @pl.when(copy_cond(kk - R))
                    def _():
                        s_out(kk - R).wait()

                @pl.when(copy_cond(kk))
                def _():
                    g_var(kk).start()

            if kk >= G:
                j = kk - G

                @pl.when(copy_cond(j))
                def _():
                    g_var(j).wait()
                    s_out(j).start()

        for j in range(NCH - R, NCH):

            @pl.when(copy_cond(j))
            def _():
                s_out(j).wait()

        nch = (ln + CH - 1) // CH
        iota = lax.iota(jnp.int32, 16)

        def g_upd(t):
            p = t % R
            return pltpu.make_async_copy(upd_hbm.at[sidx[p]], buf.at[p], sg[p])

        def s_upd(t):
            p = t % R
            return pltpu.make_async_copy(buf.at[p], out_hbm.at[didx[p]], ss[p])

        for t in range(UCH + G):
            if t < UCH:
                if t >= R:

                    @pl.when(t - R < nch)
                    def _():
                        s_upd(t - R).wait()

                @pl.when(t < nch)
                def _():
                    p = t % R
                    for b in range(CH // 16):
                        q = jnp.minimum(t * CH + b * 16 + iota, ln - 1)
                        sidx[p][pl.ds(b * 16, 16)] = ubase + q
                        didx[p][pl.ds(b * 16, 16)] = vbase + off + q
                    g_upd(t).start()

            if t >= G:
                j = t - G

                @pl.when(j < nch)
                def _():
                    g_upd(j).wait()
                    s_upd(j).start()

        for j in range(UCH - R, UCH):

            @pl.when(j < nch)
            def _():
                s_upd(j).wait()

    return k


def kernel(varRef, indice, updates, mask, reduce, axis):
    idx = indice.astype(jnp.int32)
    off = jnp.clip(idx[:, 0], 0, M)
    ln = jnp.clip(idx[:, 1], 0, M - off)
    ln = jnp.where(mask, ln, 0)
    idx3 = jnp.zeros((N, 8, 128), jnp.int32)
    idx3 = idx3.at[:, 0, 0].set(off).at[:, 0, 1].set(ln)
    out = _build_sc_kernel()(
        varRef.reshape(N * M, D), updates.reshape(N * U, D), idx3
    )
    return out.reshape(N, M, D)
